# Initial kernel scaffold; baseline (speedup 1.0000x reference)
import jax
import jax.numpy as jnp
from jax.experimental import pallas as pl


def kernel(x, edge_index, W0, b0, W1, b1, W2, b2, W3, b3, W4, b4, W5, b5, W6, b6, W7, b7):
    raise NotImplementedError("write your pallas kernel here")


# speedup vs baseline: 6.6297x; 6.6297x over previous
import functools

import jax
import jax.numpy as jnp
from jax import lax
from jax.experimental import pallas as pl
from jax.experimental.pallas import tpu as pltpu
from jax.experimental.pallas import tpu_sc as plsc

_N = 10000
_E = 320000
_NC = 2
_NS = 16
_NW = _NC * _NS
_K = 128
_CHUNKS = 79
_EPW = _K * _CHUNKS
_E_PAD = _NW * _EPW
_RPS = 640
_N_ACC = _NS * _RPS
_TRASH = _N
_BR = 400


@functools.cache
def _sc_scatter(D):
    mesh = plsc.VectorSubcoreMesh(core_axis_name="c", subcore_axis_name="s")

    @functools.partial(
        pl.kernel,
        out_type=jax.ShapeDtypeStruct((_NC, _N_ACC, D), jnp.float32),
        mesh=mesh,
        compiler_params=pltpu.CompilerParams(use_tc_tiling_on_sc=False),
        scratch_types=[
            pltpu.VMEM((_K,), jnp.int32),
            pltpu.VMEM((_K,), jnp.int32),
            pltpu.VMEM((_K, D), jnp.float32),
            pltpu.VMEM((_K, D), jnp.float32),
            pltpu.VMEM_SHARED((_N_ACC, D), jnp.float32),
            pltpu.SemaphoreType.DMA,
        ],
    )
    def scatter(rows_hbm, cols_hbm, y_hbm, out_hbm,
                rows_v, cols_v, gath_v, zero_v, acc_sh, sem):
        cid = lax.axis_index("c")
        sid = lax.axis_index("s")
        wid = sid * _NC + cid

        zvec = jnp.zeros((16,), jnp.float32)

        def zbody(i, _):
            for j in range(D // 16):
                zero_v[i, pl.ds(j * 16, 16)] = zvec
            return 0

        lax.fori_loop(0, _K, zbody, 0)
        for k in range(_RPS // _K):
            base = pl.multiple_of(sid * _RPS + k * _K, 8)
            pltpu.sync_copy(zero_v, acc_sh.at[pl.ds(base, _K)])
        plsc.subcore_barrier()

        ebase = wid * _EPW

        def body(i, _):
            off = pl.multiple_of(ebase + i * _K, 8)
            pltpu.sync_copy(rows_hbm.at[pl.ds(off, _K)], rows_v)
            pltpu.sync_copy(cols_hbm.at[pl.ds(off, _K)], cols_v)
            pltpu.async_copy(y_hbm.at[rows_v], gath_v, sem).wait()
            pltpu.sync_copy(gath_v, acc_sh.at[cols_v], add=True)
            return 0

        lax.fori_loop(0, _CHUNKS, body, 0)
        plsc.subcore_barrier()

        base = pl.multiple_of(sid * _RPS, 8)
        pltpu.sync_copy(acc_sh.at[pl.ds(base, _RPS)],
                        out_hbm.at[cid, pl.ds(base, _RPS)])

    return scatter


def _dinv_body(pp_ref, out_ref):
    deg = pp_ref[0, :, 0:1] + pp_ref[1, :, 0:1] + 1.0
    out_ref[...] = lax.rsqrt(deg)


def _dinv(pp):
    return pl.pallas_call(
        _dinv_body,
        out_shape=jax.ShapeDtypeStruct((_N_ACC, 1), jnp.float32),
    )(pp)


@functools.cache
def _tc_stage(din, has_p, dpre, has_b, act, dout, scale_out, out_scale):
    dmid = dpre if dpre else din
    dfin = dout if dout else dmid

    def body(*refs):
        refs = list(refs)
        p0 = refs.pop(0) if has_p else None
        p1 = refs.pop(0) if has_p else None
        y = refs.pop(0)
        dv = refs.pop(0)
        wpre = refs.pop(0) if dpre else None
        b = refs.pop(0) if has_b else None
        wout = refs.pop(0) if dout else None
        out = refs.pop(0)

        u = y[...]
        if has_p:
            u = p0[...] + p1[...] + u
        u = u * dv[...]
        if dpre:
            z = jnp.dot(u, wpre[...], preferred_element_type=jnp.float32)
        else:
            z = u
        if has_b:
            z = z + b[...]
        t = jnp.maximum(z, 0.0) if act else z
        if dout:
            if scale_out:
                t = t * dv[...]
            out[...] = jnp.dot(t, wout[...], preferred_element_type=jnp.float32)
        elif out_scale:
            out[...] = t * dv[...]
        else:
            out[...] = t

    row = lambda d: pl.BlockSpec((_BR, d), lambda i: (i, 0))
    full = lambda a, b_: pl.BlockSpec((a, b_), lambda i: (0, 0))
    in_specs = []
    if has_p:
        in_specs += [row(din), row(din)]
    in_specs += [row(din), pl.BlockSpec((_BR, 1), lambda i: (i, 0))]
    if dpre:
        in_specs.append(full(din, dpre))
    if has_b:
        in_specs.append(full(1, dmid))
    if dout:
        in_specs.append(full(dmid, dout))

    return pl.pallas_call(
        body,
        grid=(_N // _BR,),
        in_specs=in_specs,
        out_specs=row(dfin),
        out_shape=jax.ShapeDtypeStruct((_N, dfin), jnp.float32),
    )


def kernel(x, edge_index, W0, b0, W1, b1, W2, b2, W3, b3, W4, b4, W5, b5,
           W6, b6, W7, b7):
    rows = edge_index[0].astype(jnp.int32)
    cols = edge_index[1].astype(jnp.int32)
    rows_p = jnp.concatenate([rows, jnp.zeros((_E_PAD - _E,), jnp.int32)])
    cols_p = jnp.concatenate([cols, jnp.full((_E_PAD - _E,), _TRASH, jnp.int32)])

    ones16 = jnp.ones((_N, 16), jnp.float32)
    degp = _sc_scatter(16)(rows_p, cols_p, ones16)
    dinv = _dinv(degp)[:_N]

    def spmm(y, D):
        parts = []
        for c0 in range(0, D, 128):
            dc = min(128, D - c0)
            pc = _sc_scatter(dc)(rows_p, cols_p,
                                 y[:, c0:c0 + dc] if D > 128 else y)
            parts.append(pc[:, :_N, :])
        if len(parts) == 1:
            return parts[0][0], parts[0][1]
        cat = jnp.concatenate(parts, axis=-1)
        return cat[0], cat[1]

    bs = [b.reshape(1, -1) for b in (b0, b1, b2, b3, b4, b5, b6, b7)]
    Ws = [W0, W1, W2, W3, W4, W5, W6, W7]

    y = _tc_stage(128, False, 0, False, False, 64, False, False)(x, dinv, W0)

    dims = [64, 64, 64, 64, 128, 512, 256, 16]
    for step in range(8):
        p0, p1 = spmm(y, dims[step])
        if step in (0, 1, 5, 6):
            dnext = Ws[step + 1].shape[1]
            y = _tc_stage(dims[step], True, 0, True, True, dnext, True, False)(
                p0, p1, y, dinv, bs[step], Ws[step + 1])
        elif step == 2:
            y = _tc_stage(64, True, 0, True, True, 0, False, True)(
                p0, p1, y, dinv, bs[2])
        elif step == 3:
            y = _tc_stage(64, True, 128, True, True, 0, False, True)(
                p0, p1, y, dinv, Ws[3], bs[3])
        elif step == 4:
            y = _tc_stage(128, True, 1024, True, True, 512, True, False)(
                p0, p1, y, dinv, Ws[4], bs[4], Ws[5])
        else:
            y = _tc_stage(16, True, 0, True, False, 0, False, False)(
                p0, p1, y, dinv, bs[7])
    return y


# speedup vs baseline: 6.8193x; 1.0286x over previous
import functools

import jax
import jax.numpy as jnp
from jax import lax
from jax.experimental import pallas as pl
from jax.experimental.pallas import tpu as pltpu
from jax.experimental.pallas import tpu_sc as plsc

_N = 10000
_E = 320000
_NC = 2
_NS = 16
_NW = _NC * _NS
_K = 128
_CHUNKS = 80
_NBUF = 5
_EPW = _K * _CHUNKS
_E_PAD = _NW * _EPW
_RPS = 640
_N_ACC = _NS * _RPS
_TRASH = _N
_BR = 400


def _zero_acc(gath0, acc_sh, sid, D):
    zvec = jnp.zeros((16,), jnp.float32)

    def zbody(i, _):
        for j in range(D // 16):
            gath0[i, pl.ds(j * 16, 16)] = zvec
        return 0

    lax.fori_loop(0, _K, zbody, 0)
    for k in range(_RPS // _K):
        base = pl.multiple_of(sid * _RPS + k * _K, 8)
        pltpu.sync_copy(gath0, acc_sh.at[pl.ds(base, _K)])


def _copy_out(acc_sh, out_hbm, cid, sid):
    base = pl.multiple_of(sid * _RPS, 8)
    pltpu.sync_copy(acc_sh.at[pl.ds(base, _RPS)],
                    out_hbm.at[cid, pl.ds(base, _RPS)])


@functools.cache
def _sc_scatter(W, F):
    mesh = plsc.VectorSubcoreMesh(core_axis_name="c", subcore_axis_name="s")
    grp = _CHUNKS // _NBUF

    @functools.partial(
        pl.kernel,
        out_type=jax.ShapeDtypeStruct((F, _NC, _N_ACC, W), jnp.float32),
        mesh=mesh,
        compiler_params=pltpu.CompilerParams(use_tc_tiling_on_sc=False),
        scratch_types=[
            pltpu.VMEM((_CHUNKS, _K), jnp.int32),
            pltpu.VMEM((_CHUNKS, _K), jnp.int32),
            pltpu.VMEM((_K, W), jnp.float32),
        ] + [pltpu.VMEM((_K, W), jnp.float32) for _ in range(_NBUF)]
          + [pltpu.VMEM_SHARED((_N_ACC, W), jnp.float32)]
          + [pltpu.SemaphoreType.DMA for _ in range(_NBUF)],
    )
    def scatter(rows_hbm, cols_hbm, y_hbm, out_hbm, rows_v, cols_v, zero_v,
                *rest):
        gath = rest[:_NBUF]
        acc_sh = rest[_NBUF]
        gsem = rest[_NBUF + 1:]
        cid = lax.axis_index("c")
        sid = lax.axis_index("s")
        wid = sid * _NC + cid

        pltpu.sync_copy(rows_hbm.at[wid], rows_v)
        pltpu.sync_copy(cols_hbm.at[wid], cols_v)
        _zero_acc(zero_v, acc_sh, sid, W)

        def prime(f):
            for b in range(_NBUF):
                pltpu.async_copy(y_hbm.at[f].at[rows_v.at[b]], gath[b],
                                 gsem[b])

        prime(0)
        plsc.subcore_barrier()

        for f in range(F):
            def group(g, _, f=f):
                for b in range(_NBUF):
                    i = g * _NBUF + b
                    pltpu.make_async_copy(y_hbm.at[f].at[pl.ds(0, _K)],
                                          gath[b], gsem[b]).wait()
                    pltpu.sync_copy(gath[b], acc_sh.at[cols_v.at[i]],
                                    add=True)
                    pltpu.async_copy(y_hbm.at[f].at[rows_v.at[i + _NBUF]],
                                     gath[b], gsem[b])
                return 0

            lax.fori_loop(0, grp - 1, group, 0)
            for b in range(_NBUF):
                i = (grp - 1) * _NBUF + b
                pltpu.make_async_copy(y_hbm.at[f].at[pl.ds(0, _K)],
                                      gath[b], gsem[b]).wait()
                pltpu.sync_copy(gath[b], acc_sh.at[cols_v.at[i]], add=True)
            plsc.subcore_barrier()
            if f + 1 < F:
                prime(f + 1)
            _copy_out(acc_sh, out_hbm.at[f], cid, sid)
            if f + 1 < F:
                for k in range(_RPS // _K):
                    base = pl.multiple_of(sid * _RPS + k * _K, 8)
                    pltpu.sync_copy(zero_v, acc_sh.at[pl.ds(base, _K)])
                plsc.subcore_barrier()

    return scatter


@functools.cache
def _sc_degree():
    D = 16
    mesh = plsc.VectorSubcoreMesh(core_axis_name="c", subcore_axis_name="s")

    @functools.partial(
        pl.kernel,
        out_type=jax.ShapeDtypeStruct((_NC, _N_ACC, D), jnp.float32),
        mesh=mesh,
        compiler_params=pltpu.CompilerParams(use_tc_tiling_on_sc=False),
        scratch_types=[
            pltpu.VMEM((_CHUNKS, _K), jnp.int32),
            pltpu.VMEM((_K, D), jnp.float32),
            pltpu.VMEM((_K, D), jnp.float32),
            pltpu.VMEM_SHARED((_N_ACC, D), jnp.float32),
        ],
    )
    def degree(cols_hbm, out_hbm, cols_v, ones_v, zero_v, acc_sh):
        cid = lax.axis_index("c")
        sid = lax.axis_index("s")
        wid = sid * _NC + cid

        pltpu.sync_copy(cols_hbm.at[wid], cols_v)
        _zero_acc(zero_v, acc_sh, sid, D)
        ovec = jnp.ones((16,), jnp.float32)

        def obody(i, _):
            ones_v[i, pl.ds(0, 16)] = ovec
            return 0

        lax.fori_loop(0, _K, obody, 0)
        plsc.subcore_barrier()

        def body(i, _):
            pltpu.sync_copy(ones_v, acc_sh.at[cols_v.at[i]], add=True)
            return 0

        lax.fori_loop(0, _CHUNKS, body, 0)
        plsc.subcore_barrier()
        _copy_out(acc_sh, out_hbm, cid, sid)

    return degree


def _dinv_body(pp_ref, out_ref):
    deg = pp_ref[0, :, 0:1] + pp_ref[1, :, 0:1] + 1.0
    out_ref[...] = lax.rsqrt(deg)


def _dinv(pp):
    return pl.pallas_call(
        _dinv_body,
        out_shape=jax.ShapeDtypeStruct((_N_ACC, 1), jnp.float32),
    )(pp)


@functools.cache
def _tc_stage(W, F, has_p, dpre, has_b, act, dout, scale_out, out_scale,
              fout):
    din = F * W
    dmid = dpre if dpre else din
    dfin = dout if dout else dmid

    def body(*refs):
        refs = list(refs)
        pc = refs.pop(0) if has_p else None
        y = refs.pop(0)
        dv = refs.pop(0)
        wpre = refs.pop(0) if dpre else None
        b = refs.pop(0) if has_b else None
        wout = refs.pop(0) if dout else None
        out = refs.pop(0)

        if F > 1:
            yb = [y[f] for f in range(F)]
        else:
            yb = [y[...]]
        if has_p:
            chunks = [pc[f, 0] + pc[f, 1] + yb[f] for f in range(F)]
        else:
            chunks = yb
        u = chunks[0] if F == 1 else jnp.concatenate(chunks, axis=-1)
        u = u * dv[...]
        if dpre:
            z = jnp.dot(u, wpre[...], preferred_element_type=jnp.float32)
        else:
            z = u
        if has_b:
            z = z + b[...]
        t = jnp.maximum(z, 0.0) if act else z
        if dout:
            if scale_out:
                t = t * dv[...]
            res = jnp.dot(t, wout[...], preferred_element_type=jnp.float32)
        elif out_scale:
            res = t * dv[...]
        else:
            res = t
        if fout:
            w_ = dfin // fout
            for f in range(fout):
                out[f] = res[:, f * w_:(f + 1) * w_]
        else:
            out[...] = res

    full = lambda a, b_: pl.BlockSpec((a, b_), lambda i: (0, 0))
    in_specs = []
    if has_p:
        in_specs.append(pl.BlockSpec((F, 2, _BR, W), lambda i: (0, 0, i, 0)))
    if F > 1:
        in_specs.append(pl.BlockSpec((F, _BR, W), lambda i: (0, i, 0)))
    else:
        in_specs.append(pl.BlockSpec((_BR, din), lambda i: (i, 0)))
    in_specs.append(pl.BlockSpec((_BR, 1), lambda i: (i, 0)))
    if dpre:
        in_specs.append(full(din, dpre))
    if has_b:
        in_specs.append(full(1, dmid))
    if dout:
        in_specs.append(full(dmid, dout))
    if fout:
        out_spec = pl.BlockSpec((fout, _BR, dfin // fout),
                                lambda i: (0, i, 0))
        out_shape = jax.ShapeDtypeStruct((fout, _N, dfin // fout),
                                         jnp.float32)
    else:
        out_spec = pl.BlockSpec((_BR, dfin), lambda i: (i, 0))
        out_shape = jax.ShapeDtypeStruct((_N, dfin), jnp.float32)

    return pl.pallas_call(
        body,
        grid=(_N // _BR,),
        in_specs=in_specs,
        out_specs=out_spec,
        out_shape=out_shape,
    )


def kernel(x, edge_index, W0, b0, W1, b1, W2, b2, W3, b3, W4, b4, W5, b5,
           W6, b6, W7, b7):
    rows = edge_index[0].astype(jnp.int32)
    cols = edge_index[1].astype(jnp.int32)
    rows_p = jnp.concatenate([rows, jnp.zeros((_E_PAD - _E,), jnp.int32)])
    cols_p = jnp.concatenate([cols, jnp.full((_E_PAD - _E,), _TRASH, jnp.int32)])
    rows_p = rows_p.reshape(_NW, _CHUNKS, _K)
    cols_p = cols_p.reshape(_NW, _CHUNKS, _K)

    degp = _sc_degree()(cols_p)
    dinv = _dinv(degp)

    def spmm(yf):
        F, _, W = yf.shape
        return _sc_scatter(W, F)(rows_p, cols_p, yf)

    bs = [b.reshape(1, -1) for b in (b0, b1, b2, b3, b4, b5, b6, b7)]

    y = _tc_stage(128, 1, False, 0, False, False, 64, False, False, 0)(
        x, dinv, W0)

    pc = spmm(y[None])
    y = _tc_stage(64, 1, True, 0, True, True, 64, True, False, 0)(
        pc, y, dinv, bs[0], W1)
    pc = spmm(y[None])
    y = _tc_stage(64, 1, True, 0, True, True, 64, True, False, 0)(
        pc, y, dinv, bs[1], W2)
    pc = spmm(y[None])
    y = _tc_stage(64, 1, True, 0, True, True, 0, False, True, 0)(
        pc, y, dinv, bs[2])
    pc = spmm(y[None])
    y = _tc_stage(64, 1, True, 128, True, True, 0, False, True, 2)(
        pc, y, dinv, W3, bs[3])
    pc = spmm(y)
    y = _tc_stage(64, 2, True, 1024, True, True, 512, True, False, 8)(
        pc, y, dinv, W4, bs[4], W5)
    pc = spmm(y)
    y = _tc_stage(64, 8, True, 0, True, True, 256, True, False, 4)(
        pc, y, dinv, bs[5], W6)
    pc = spmm(y)
    y = _tc_stage(64, 4, True, 0, True, True, 16, True, False, 0)(
        pc, y, dinv, bs[6], W7)
    pc = spmm(y[None])
    return _tc_stage(16, 1, True, 0, True, False, 0, False, False, 0)(
        pc, y, dinv, bs[7])


# speedup vs baseline: 22.1544x; 3.2488x over previous
import functools

import jax
import jax.numpy as jnp
from jax import lax
from jax.experimental import pallas as pl
from jax.experimental.pallas import tpu as pltpu
from jax.experimental.pallas import tpu_sc as plsc

_N = 10000
_E = 320000
_NC = 2
_NS = 16
_NW = _NC * _NS
_K = 128
_CHUNKS = 80
_NBUF = 5
_EPW = _K * _CHUNKS
_E_PAD = _NW * _EPW
_RPS = 640
_N_ACC = _NS * _RPS
_TRASH = _N
_BR = 400


def _zero_acc(gath0, acc_sh, sid, D):
    zvec = jnp.zeros((16,), jnp.float32)

    def zbody(i, _):
        for j in range(D // 16):
            gath0[i, pl.ds(j * 16, 16)] = zvec
        return 0

    lax.fori_loop(0, _K, zbody, 0)
    for k in range(_RPS // _K):
        base = pl.multiple_of(sid * _RPS + k * _K, 8)
        pltpu.sync_copy(gath0, acc_sh.at[pl.ds(base, _K)])


def _copy_out(acc_sh, out_hbm, cid, sid):
    base = pl.multiple_of(sid * _RPS, 8)
    pltpu.sync_copy(acc_sh.at[pl.ds(base, _RPS)],
                    out_hbm.at[cid, pl.ds(base, _RPS)])


@functools.cache
def _sc_scatter(W, F):
    mesh = plsc.VectorSubcoreMesh(core_axis_name="c", subcore_axis_name="s")
    grp = _CHUNKS // _NBUF

    @functools.partial(
        pl.kernel,
        out_type=jax.ShapeDtypeStruct((F, _NC, _N_ACC, W), jnp.float32),
        mesh=mesh,
        compiler_params=pltpu.CompilerParams(use_tc_tiling_on_sc=False),
        scratch_types=[
            pltpu.VMEM((_CHUNKS, _K), jnp.int32),
            pltpu.VMEM((_CHUNKS, _K), jnp.int32),
            pltpu.VMEM((_K, W), jnp.float32),
        ] + [pltpu.VMEM((_K, W), jnp.float32) for _ in range(_NBUF)]
          + [pltpu.VMEM_SHARED((_N_ACC, W), jnp.float32)]
          + [pltpu.SemaphoreType.DMA for _ in range(_NBUF)],
    )
    def scatter(rows_hbm, cols_hbm, y_hbm, out_hbm, rows_v, cols_v, zero_v,
                *rest):
        gath = rest[:_NBUF]
        acc_sh = rest[_NBUF]
        gsem = rest[_NBUF + 1:]
        cid = lax.axis_index("c")
        sid = lax.axis_index("s")
        wid = sid * _NC + cid

        pltpu.sync_copy(rows_hbm.at[wid], rows_v)
        pltpu.sync_copy(cols_hbm.at[wid], cols_v)
        _zero_acc(zero_v, acc_sh, sid, W)

        def issue(f, i, b):
            pltpu.async_copy(y_hbm.at[f].at[rows_v.at[i]], gath[b], gsem[b])

        def wait(f, i, b):
            pltpu.make_async_copy(y_hbm.at[f].at[pl.ds(0, _K)], gath[b],
                                  gsem[b]).wait()

        def scat(i, b):
            pltpu.sync_copy(gath[b], acc_sh.at[cols_v.at[i]], add=True)

        for b in range(_NBUF):
            issue(0, b, b)
        plsc.subcore_barrier()

        for f in range(F):
            def group(g, _, f=f):
                for b in range(_NBUF):
                    i = g * _NBUF + b
                    wait(f, i, b)
                    scat(i, b)
                    issue(f, i + _NBUF, b)
                return 0

            lax.fori_loop(0, grp - 1, group, 0)
            for b in range(_NBUF):
                i = (grp - 1) * _NBUF + b
                wait(f, i, b)
                scat(i, b)
            plsc.subcore_barrier()
            if f + 1 < F:
                for b in range(_NBUF):
                    issue(f + 1, b, b)
            _copy_out(acc_sh, out_hbm.at[f], cid, sid)
            if f + 1 < F:
                for k in range(_RPS // _K):
                    base = pl.multiple_of(sid * _RPS + k * _K, 8)
                    pltpu.sync_copy(zero_v, acc_sh.at[pl.ds(base, _K)])
                plsc.subcore_barrier()

    return scatter


@functools.cache
def _sc_degree():
    D = 16
    mesh = plsc.VectorSubcoreMesh(core_axis_name="c", subcore_axis_name="s")

    @functools.partial(
        pl.kernel,
        out_type=jax.ShapeDtypeStruct((_NC, _N_ACC, D), jnp.float32),
        mesh=mesh,
        compiler_params=pltpu.CompilerParams(use_tc_tiling_on_sc=False),
        scratch_types=[
            pltpu.VMEM((_CHUNKS, _K), jnp.int32),
            pltpu.VMEM((_K, D), jnp.float32),
            pltpu.VMEM((_K, D), jnp.float32),
            pltpu.VMEM_SHARED((_N_ACC, D), jnp.float32),
        ],
    )
    def degree(cols_hbm, out_hbm, cols_v, ones_v, zero_v, acc_sh):
        cid = lax.axis_index("c")
        sid = lax.axis_index("s")
        wid = sid * _NC + cid

        pltpu.sync_copy(cols_hbm.at[wid], cols_v)
        _zero_acc(zero_v, acc_sh, sid, D)
        ovec = jnp.ones((16,), jnp.float32)

        def obody(i, _):
            ones_v[i, pl.ds(0, 16)] = ovec
            return 0

        lax.fori_loop(0, _K, obody, 0)
        plsc.subcore_barrier()

        def body(i, _):
            pltpu.sync_copy(ones_v, acc_sh.at[cols_v.at[i]], add=True)
            return 0

        lax.fori_loop(0, _CHUNKS, body, 0)
        plsc.subcore_barrier()
        _copy_out(acc_sh, out_hbm, cid, sid)

    return degree


def _dinv_body(pp_ref, out_ref):
    deg = pp_ref[0, :, 0:1] + pp_ref[1, :, 0:1] + 1.0
    out_ref[...] = lax.rsqrt(deg)


def _dinv(pp):
    return pl.pallas_call(
        _dinv_body,
        out_shape=jax.ShapeDtypeStruct((_N_ACC, 1), jnp.float32),
    )(pp)


@functools.cache
def _tc_stage(W, F, has_p, dpre, has_b, act, dout, scale_out, out_scale,
              fout):
    din = F * W
    dmid = dpre if dpre else din
    dfin = dout if dout else dmid

    def body(*refs):
        refs = list(refs)
        pc = refs.pop(0) if has_p else None
        y = refs.pop(0)
        dv = refs.pop(0)
        wpre = refs.pop(0) if dpre else None
        b = refs.pop(0) if has_b else None
        wout = refs.pop(0) if dout else None
        out = refs.pop(0)

        if F > 1:
            yb = [y[f] for f in range(F)]
        else:
            yb = [y[...]]
        if has_p:
            chunks = [pc[f, 0] + pc[f, 1] + yb[f] for f in range(F)]
        else:
            chunks = yb
        u = chunks[0] if F == 1 else jnp.concatenate(chunks, axis=-1)
        u = u * dv[...]
        if dpre:
            z = jnp.dot(u, wpre[...], preferred_element_type=jnp.float32)
        else:
            z = u
        if has_b:
            z = z + b[...]
        t = jnp.maximum(z, 0.0) if act else z
        if dout:
            if scale_out:
                t = t * dv[...]
            res = jnp.dot(t, wout[...], preferred_element_type=jnp.float32)
        elif out_scale:
            res = t * dv[...]
        else:
            res = t
        if fout:
            w_ = dfin // fout
            for f in range(fout):
                out[f] = res[:, f * w_:(f + 1) * w_]
        else:
            out[...] = res

    full = lambda a, b_: pl.BlockSpec((a, b_), lambda i: (0, 0))
    in_specs = []
    if has_p:
        in_specs.append(pl.BlockSpec((F, 2, _BR, W), lambda i: (0, 0, i, 0)))
    if F > 1:
        in_specs.append(pl.BlockSpec((F, _BR, W), lambda i: (0, i, 0)))
    else:
        in_specs.append(pl.BlockSpec((_BR, din), lambda i: (i, 0)))
    in_specs.append(pl.BlockSpec((_BR, 1), lambda i: (i, 0)))
    if dpre:
        in_specs.append(full(din, dpre))
    if has_b:
        in_specs.append(full(1, dmid))
    if dout:
        in_specs.append(full(dmid, dout))
    if fout:
        out_spec = pl.BlockSpec((fout, _BR, dfin // fout),
                                lambda i: (0, i, 0))
        out_shape = jax.ShapeDtypeStruct((fout, _N, dfin // fout),
                                         jnp.float32)
    else:
        out_spec = pl.BlockSpec((_BR, dfin), lambda i: (i, 0))
        out_shape = jax.ShapeDtypeStruct((_N, dfin), jnp.float32)

    return pl.pallas_call(
        body,
        grid=(_N // _BR,),
        in_specs=in_specs,
        out_specs=out_spec,
        out_shape=out_shape,
    )


def kernel(x, edge_index, W0, b0, W1, b1, W2, b2, W3, b3, W4, b4, W5, b5,
           W6, b6, W7, b7):
    rows = edge_index[0].astype(jnp.int32)
    cols = edge_index[1].astype(jnp.int32)
    npad = _E_PAD - _E
    pad_i = jnp.arange(npad, dtype=jnp.int32)
    rows_p = jnp.concatenate([rows, pad_i % _N])
    cols_p = jnp.concatenate([cols, _N + pad_i % (_N_ACC - _N)])
    rows_p = rows_p.reshape(_NW, _CHUNKS, _K)
    cols_p = cols_p.reshape(_NW, _CHUNKS, _K)

    degp = _sc_degree()(cols_p)
    dinv = _dinv(degp)

    def spmm(yf):
        F, _, W = yf.shape
        return _sc_scatter(W, F)(rows_p, cols_p, yf)

    bs = [b.reshape(1, -1) for b in (b0, b1, b2, b3, b4, b5, b6, b7)]

    y = _tc_stage(128, 1, False, 0, False, False, 64, False, False, 0)(
        x, dinv, W0)

    pc = spmm(y[None])
    y = _tc_stage(64, 1, True, 0, True, True, 64, True, False, 0)(
        pc, y, dinv, bs[0], W1)
    pc = spmm(y[None])
    y = _tc_stage(64, 1, True, 0, True, True, 64, True, False, 0)(
        pc, y, dinv, bs[1], W2)
    pc = spmm(y[None])
    y = _tc_stage(64, 1, True, 0, True, True, 0, False, True, 0)(
        pc, y, dinv, bs[2])
    pc = spmm(y[None])
    y = _tc_stage(64, 1, True, 128, True, True, 0, False, True, 2)(
        pc, y, dinv, W3, bs[3])
    pc = spmm(y)
    y = _tc_stage(64, 2, True, 1024, True, True, 512, True, False, 8)(
        pc, y, dinv, W4, bs[4], W5)
    pc = spmm(y)
    y = _tc_stage(64, 8, True, 0, True, True, 256, True, False, 4)(
        pc, y, dinv, bs[5], W6)
    pc = spmm(y)
    y = _tc_stage(64, 4, True, 0, True, True, 16, True, False, 0)(
        pc, y, dinv, bs[6], W7)
    pc = spmm(y[None])
    return _tc_stage(16, 1, True, 0, True, False, 0, False, False, 0)(
        pc, y, dinv, bs[7])
